# Initial kernel scaffold; baseline (speedup 1.0000x reference)
import jax
import jax.numpy as jnp
from jax.experimental import pallas as pl


def kernel(r, K, mask, Wq, bq, Wk, bk, Wv, bv, Wo, bo):
    raise NotImplementedError("write your pallas kernel here")


# speedup vs baseline: 4.2756x; 4.2756x over previous
import jax
import jax.numpy as jnp
from jax.experimental import pallas as pl
from jax.experimental.pallas import tpu as pltpu

D = 512
H = 8
DK = D // H
INV_SQRT_DK = 1.0 / (DK ** 0.5)


def _pool_kernel(r_ref, k_ref, mask_ref, wq_ref, bq_ref, wk_ref, bk_ref,
                 wv_ref, bv_ref, wo_ref, bo_ref, attn_ref, pooled_ref):
    f32 = jnp.float32
    q = jax.lax.dot_general(
        r_ref[0], wq_ref[...], (((1,), (1,)), ((), ())),
        preferred_element_type=f32) + bq_ref[...]
    h_ids = jax.lax.broadcasted_iota(jnp.int32, (H, D), 0)
    d_ids = jax.lax.broadcasted_iota(jnp.int32, (H, D), 1)
    hm = (d_ids // DK) == h_ids
    m8 = jnp.where(hm, jnp.broadcast_to(q, (H, D)), f32(0.0))
    a_s = jax.lax.dot_general(
        m8, wk_ref[...], (((1,), (0,)), ((), ())),
        preferred_element_type=f32) * f32(INV_SQRT_DK)
    c_s = jnp.sum(m8 * bk_ref[...], axis=1, keepdims=True) * f32(INV_SQRT_DK)

    kb = k_ref[0]
    s = jax.lax.dot_general(
        a_s, kb, (((1,), (1,)), ((), ())),
        preferred_element_type=f32) + c_s
    mrow = mask_ref[0]
    s = jnp.where(mrow != f32(0.0), s, f32(-1e9))

    m = jnp.max(s, axis=1, keepdims=True)
    p = jnp.exp(s - m)
    l = jnp.sum(p, axis=1, keepdims=True)
    rl = f32(1.0) / l
    attn_ref[0] = p * rl

    pn = jax.lax.dot_general(
        p, kb, (((1,), (0,)), ((), ())),
        preferred_element_type=f32) * rl
    g = jax.lax.dot_general(
        pn, wv_ref[...], (((1,), (1,)), ((), ())),
        preferred_element_type=f32)
    pooled = jnp.sum(jnp.where(hm, g, f32(0.0)), axis=0, keepdims=True)
    pooled = pooled + bv_ref[...]
    out = jax.lax.dot_general(
        pooled, wo_ref[...], (((1,), (1,)), ((), ())),
        preferred_element_type=f32) + bo_ref[...]
    pooled_ref[0] = out


def kernel(r, K, mask, Wq, bq, Wk, bk, Wv, bv, Wo, bo):
    B, R, d = K.shape
    r3 = r.reshape(B, 1, d)
    mask3 = mask.astype(jnp.float32).reshape(B, 1, R)
    b2 = [b.reshape(1, d) for b in (bq, bk, bv, bo)]

    wspec = pl.BlockSpec((d, d), lambda b: (0, 0))
    bspec = pl.BlockSpec((1, d), lambda b: (0, 0))
    attn, pooled3 = pl.pallas_call(
        _pool_kernel,
        grid=(B,),
        in_specs=[
            pl.BlockSpec((1, 1, d), lambda b: (b, 0, 0)),
            pl.BlockSpec((1, R, d), lambda b: (b, 0, 0)),
            pl.BlockSpec((1, 1, R), lambda b: (b, 0, 0)),
            wspec, bspec,
            wspec, bspec,
            wspec, bspec,
            wspec, bspec,
        ],
        out_specs=[
            pl.BlockSpec((1, H, R), lambda b: (b, 0, 0)),
            pl.BlockSpec((1, 1, d), lambda b: (b, 0, 0)),
        ],
        out_shape=[
            jax.ShapeDtypeStruct((B, H, R), jnp.float32),
            jax.ShapeDtypeStruct((B, 1, d), jnp.float32),
        ],
        compiler_params=pltpu.CompilerParams(
            dimension_semantics=("parallel",),
            vmem_limit_bytes=50 * 1024 * 1024,
        ),
        name="cross_attention_pool",
    )(r3, K, mask3, Wq, b2[0], Wk, b2[1], Wv, b2[2], Wo, b2[3])
    return (pooled3.reshape(B, d), attn)
